# baseline (speedup 1.0000x reference)
import functools

import jax
import jax.numpy as jnp
from jax import lax
from jax.experimental import pallas as pl
from jax.experimental.pallas import tpu as pltpu, tpu_sc as plsc

VOCAB = 1000
VPAD = 1024
B, T = 1024, 50
TPAD = 56
N_TOK = B * T
LSE_PAD = 1024

NC, NS = 2, 16
NW = NC * NS
TOK_PER_W = N_TOK // NW
BATCH_PER_W = B // NW
NBUF = 2
L = 16
N_GROUPS = TOK_PER_W // L


def _row_lse_body(table_ref, out_ref):
    t = table_ref[...]
    m = jnp.max(t, axis=1, keepdims=True)
    s = jnp.sum(jnp.exp(t - m), axis=1, keepdims=True)
    out_ref[0:VOCAB, :] = m + jnp.log(s)


def _row_lse(table):
    out = pl.pallas_call(
        _row_lse_body,
        out_shape=jax.ShapeDtypeStruct((LSE_PAD, 1), jnp.float32),
    )(table)
    return out.reshape(LSE_PAD)


def _gather_body(idxp_hbm, tabpad_hbm, out_hbm, idxp_v, rows_v, gsem, ssem):
    cid = lax.axis_index("c")
    sid = lax.axis_index("s")
    wid = sid * NC + cid
    bbase = wid * BATCH_PER_W

    pltpu.sync_copy(
        idxp_hbm.at[pl.ds(bbase * TPAD, BATCH_PER_W * TPAD)], idxp_v)

    def gather_desc(g, b):
        return pltpu.make_async_copy(
            tabpad_hbm.at[idxp_v.at[pl.ds(g * TPAD, TPAD)]],
            rows_v.at[b], gsem)

    def scatter_desc(g, b):
        return pltpu.make_async_copy(rows_v.at[b], out_hbm.at[bbase + g], ssem)

    gather_desc(0, 0).start()

    def step(g, _):
        b = lax.rem(g, NBUF)
        gather_desc(g, b).wait()

        @pl.when(g >= 1)
        def _():
            scatter_desc(g - 1, 1 - b).wait()

        scatter_desc(g, b).start()

        @pl.when(g + 1 < BATCH_PER_W)
        def _():
            gather_desc(g + 1, 1 - b).start()

        return 0

    lax.fori_loop(0, BATCH_PER_W, step, 0)
    scatter_desc(BATCH_PER_W - 1, (BATCH_PER_W - 1) % NBUF).wait()


@functools.partial(
    pl.kernel,
    out_type=jax.ShapeDtypeStruct((B, TPAD, VOCAB), jnp.float32),
    mesh=plsc.VectorSubcoreMesh(core_axis_name="c", subcore_axis_name="s"),
    compiler_params=pltpu.CompilerParams(
        needs_layout_passes=False, use_tc_tiling_on_sc=False),
    scratch_types=[
        pltpu.VMEM((BATCH_PER_W * TPAD,), jnp.int32),
        pltpu.VMEM((NBUF, TPAD, VOCAB), jnp.float32),
        pltpu.SemaphoreType.DMA,
        pltpu.SemaphoreType.DMA,
    ],
)
def _sc_gather(idxp_hbm, tabpad_hbm, out_hbm, idxp_v, rows_v, gsem, ssem):
    _gather_body(idxp_hbm, tabpad_hbm, out_hbm, idxp_v, rows_v, gsem, ssem)


def _loss_body(idx_hbm, tgt_hbm, lse_hbm, tabflat_hbm, part_hbm,
               idx_v, tgt_v, comb_v, picked_v, lse_v, acc_v, psem):
    cid = lax.axis_index("c")
    sid = lax.axis_index("s")
    wid = sid * NC + cid
    base = wid * TOK_PER_W

    pltpu.sync_copy(idx_hbm.at[pl.ds(base, TOK_PER_W)], idx_v)
    pltpu.sync_copy(tgt_hbm.at[pl.ds(base, TOK_PER_W)], tgt_v)
    pltpu.sync_copy(lse_hbm, lse_v)

    def comb_step(i, _):
        o = i * L
        comb_v[pl.ds(o, L)] = idx_v[pl.ds(o, L)] * VOCAB + tgt_v[pl.ds(o, L)]
        return 0
    lax.fori_loop(0, N_GROUPS, comb_step, 0)

    pltpu.async_copy(tabflat_hbm.at[comb_v], picked_v, psem).wait()

    def loss_step(i, acc):
        o = i * L
        idxv = idx_v[pl.ds(o, L)]
        lsev = plsc.load_gather(lse_v, [idxv])
        return acc + (lsev - picked_v[pl.ds(o, L)])

    acc = lax.fori_loop(0, N_GROUPS, loss_step, jnp.zeros((L,), jnp.float32))
    acc_v[...] = acc
    pltpu.sync_copy(acc_v, part_hbm.at[wid])


@functools.partial(
    pl.kernel,
    out_type=jax.ShapeDtypeStruct((NW, L), jnp.float32),
    mesh=plsc.VectorSubcoreMesh(core_axis_name="c", subcore_axis_name="s"),
    compiler_params=pltpu.CompilerParams(
        needs_layout_passes=False, use_tc_tiling_on_sc=False),
    scratch_types=[
        pltpu.VMEM((TOK_PER_W,), jnp.int32),
        pltpu.VMEM((TOK_PER_W,), jnp.int32),
        pltpu.VMEM((TOK_PER_W,), jnp.int32),
        pltpu.VMEM((TOK_PER_W,), jnp.float32),
        pltpu.VMEM((LSE_PAD,), jnp.float32),
        pltpu.VMEM((L,), jnp.float32),
        pltpu.SemaphoreType.DMA,
    ],
)
def _sc_loss(idx_hbm, tgt_hbm, lse_hbm, tabflat_hbm, part_hbm,
             idx_v, tgt_v, comb_v, picked_v, lse_v, acc_v, psem):
    _loss_body(idx_hbm, tgt_hbm, lse_hbm, tabflat_hbm, part_hbm,
               idx_v, tgt_v, comb_v, picked_v, lse_v, acc_v, psem)


def kernel(idx, targets, table):
    idx32 = idx.astype(jnp.int32)
    idx_flat = idx32.reshape(N_TOK)
    tgt_flat = targets.reshape(N_TOK).astype(jnp.int32)

    lse = _row_lse(table)
    tabflat = jnp.concatenate(
        [table.reshape(VOCAB * VOCAB), jnp.zeros((8,), jnp.float32)])
    parts = _sc_loss(idx_flat, tgt_flat, lse, tabflat)
    loss = jnp.sum(parts) / jnp.float32(N_TOK)

    idxp = jnp.pad(idx32, ((0, 0), (0, TPAD - T))).reshape(B * TPAD)
    padded = _sc_gather(idxp, table)
    logits = padded[:, :T, :]
    return (logits, loss)


# speedup vs baseline: 1.0012x; 1.0012x over previous
import functools

import jax
import jax.numpy as jnp
from jax import lax
from jax.experimental import pallas as pl
from jax.experimental.pallas import tpu as pltpu, tpu_sc as plsc

VOCAB = 1000
VPAD = 1024
B, T = 1024, 50
TPAD = 56
N_TOK = B * T
LSE_PAD = 1024

NC, NS = 2, 16
NW = NC * NS
TOK_PER_W = N_TOK // NW
BATCH_PER_W = B // NW
NBUF = 2
L = 16
N_GROUPS = TOK_PER_W // L


def _row_lse_body(table_ref, out_ref):
    t = table_ref[...]
    m = jnp.max(t, axis=1, keepdims=True)
    s = jnp.sum(jnp.exp(t - m), axis=1, keepdims=True)
    out_ref[0:VOCAB, :] = m + jnp.log(s)


def _row_lse(table):
    out = pl.pallas_call(
        _row_lse_body,
        out_shape=jax.ShapeDtypeStruct((LSE_PAD, 1), jnp.float32),
    )(table)
    return out.reshape(LSE_PAD)


def _gather_body(idxp_hbm, tabpad_hbm, out_hbm, idxp_v, rows_v, gsem, ssem):
    cid = lax.axis_index("c")
    sid = lax.axis_index("s")
    wid = sid * NC + cid
    bbase = wid * BATCH_PER_W

    pltpu.sync_copy(idxp_hbm.at[pl.ds(bbase, BATCH_PER_W)], idxp_v)

    def gather_desc(g, b):
        return pltpu.make_async_copy(
            tabpad_hbm.at[idxp_v.at[g]], rows_v.at[b], gsem)

    def scatter_desc(g, b):
        return pltpu.make_async_copy(rows_v.at[b], out_hbm.at[bbase + g], ssem)

    gather_desc(0, 0).start()

    def step(g, _):
        b = lax.rem(g, NBUF)
        gather_desc(g, b).wait()

        @pl.when(g >= 1)
        def _():
            scatter_desc(g - 1, 1 - b).wait()

        scatter_desc(g, b).start()

        @pl.when(g + 1 < BATCH_PER_W)
        def _():
            gather_desc(g + 1, 1 - b).start()

        return 0

    lax.fori_loop(0, BATCH_PER_W, step, 0)
    scatter_desc(BATCH_PER_W - 1, (BATCH_PER_W - 1) % NBUF).wait()


@functools.partial(
    pl.kernel,
    out_type=jax.ShapeDtypeStruct((B, TPAD, VOCAB), jnp.float32),
    mesh=plsc.VectorSubcoreMesh(core_axis_name="c", subcore_axis_name="s"),
    compiler_params=pltpu.CompilerParams(
        needs_layout_passes=False, use_tc_tiling_on_sc=False),
    scratch_types=[
        pltpu.VMEM((BATCH_PER_W, TPAD), jnp.int32),
        pltpu.VMEM((NBUF, TPAD, VOCAB), jnp.float32),
        pltpu.SemaphoreType.DMA,
        pltpu.SemaphoreType.DMA,
    ],
)
def _sc_gather(idxp_hbm, tabpad_hbm, out_hbm, idxp_v, rows_v, gsem, ssem):
    _gather_body(idxp_hbm, tabpad_hbm, out_hbm, idxp_v, rows_v, gsem, ssem)


def _loss_body(idx_hbm, tgt_hbm, lse_hbm, tabflat_hbm, part_hbm,
               idx_v, tgt_v, comb_v, picked_v, lse_v, acc_v, psem):
    cid = lax.axis_index("c")
    sid = lax.axis_index("s")
    wid = sid * NC + cid
    base = wid * TOK_PER_W

    pltpu.sync_copy(idx_hbm.at[pl.ds(base, TOK_PER_W)], idx_v)
    pltpu.sync_copy(tgt_hbm.at[pl.ds(base, TOK_PER_W)], tgt_v)
    pltpu.sync_copy(lse_hbm, lse_v)

    def comb_step(i, _):
        o = i * L
        comb_v[pl.ds(o, L)] = idx_v[pl.ds(o, L)] * VOCAB + tgt_v[pl.ds(o, L)]
        return 0
    lax.fori_loop(0, N_GROUPS, comb_step, 0)

    pltpu.async_copy(tabflat_hbm.at[comb_v], picked_v, psem).wait()

    def loss_step(i, acc):
        o = i * L
        idxv = idx_v[pl.ds(o, L)]
        lsev = plsc.load_gather(lse_v, [idxv])
        return acc + (lsev - picked_v[pl.ds(o, L)])

    acc = lax.fori_loop(0, N_GROUPS, loss_step, jnp.zeros((L,), jnp.float32))
    acc_v[...] = acc
    pltpu.sync_copy(acc_v, part_hbm.at[wid])


@functools.partial(
    pl.kernel,
    out_type=jax.ShapeDtypeStruct((NW, L), jnp.float32),
    mesh=plsc.VectorSubcoreMesh(core_axis_name="c", subcore_axis_name="s"),
    compiler_params=pltpu.CompilerParams(
        needs_layout_passes=False, use_tc_tiling_on_sc=False),
    scratch_types=[
        pltpu.VMEM((TOK_PER_W,), jnp.int32),
        pltpu.VMEM((TOK_PER_W,), jnp.int32),
        pltpu.VMEM((TOK_PER_W,), jnp.int32),
        pltpu.VMEM((TOK_PER_W,), jnp.float32),
        pltpu.VMEM((LSE_PAD,), jnp.float32),
        pltpu.VMEM((L,), jnp.float32),
        pltpu.SemaphoreType.DMA,
    ],
)
def _sc_loss(idx_hbm, tgt_hbm, lse_hbm, tabflat_hbm, part_hbm,
             idx_v, tgt_v, comb_v, picked_v, lse_v, acc_v, psem):
    _loss_body(idx_hbm, tgt_hbm, lse_hbm, tabflat_hbm, part_hbm,
               idx_v, tgt_v, comb_v, picked_v, lse_v, acc_v, psem)


def kernel(idx, targets, table):
    idx32 = idx.astype(jnp.int32)
    idx_flat = idx32.reshape(N_TOK)
    tgt_flat = targets.reshape(N_TOK).astype(jnp.int32)

    lse = _row_lse(table)
    tabflat = jnp.concatenate(
        [table.reshape(VOCAB * VOCAB), jnp.zeros((8,), jnp.float32)])
    parts = _sc_loss(idx_flat, tgt_flat, lse, tabflat)
    loss = jnp.sum(parts) / jnp.float32(N_TOK)

    idxp = jnp.pad(idx32, ((0, 0), (0, TPAD - T)))
    padded = _sc_gather(idxp, table)
    logits = padded[:, :T, :]
    return (logits, loss)


# speedup vs baseline: 1.7178x; 1.7157x over previous
import functools

import jax
import jax.numpy as jnp
from jax import lax
from jax.experimental import pallas as pl
from jax.experimental.pallas import tpu as pltpu, tpu_sc as plsc

VOCAB = 1000
VPAD = 1024
B, T = 1024, 50
TPAD = 56
N_TOK = B * T
LSE_PAD = 1024

NC, NS = 2, 16
NW = NC * NS
TOK_PER_W = N_TOK // NW
BATCH_PER_W = B // NW
NBUF = 2
L = 16
N_GROUPS = TOK_PER_W // L


def _row_lse_body(table_ref, out_ref):
    t = table_ref[...]
    m = jnp.max(t, axis=1, keepdims=True)
    s = jnp.sum(jnp.exp(t - m), axis=1, keepdims=True)
    out_ref[0:VOCAB, :] = m + jnp.log(s)


def _row_lse(table):
    out = pl.pallas_call(
        _row_lse_body,
        out_shape=jax.ShapeDtypeStruct((LSE_PAD, 1), jnp.float32),
    )(table)
    return out.reshape(LSE_PAD)


def _gather_body(idxp_hbm, tabpad_hbm, out_hbm, idxp_v, rows_v, gsem, ssem):
    cid = lax.axis_index("c")
    sid = lax.axis_index("s")
    wid = sid * NC + cid
    bbase = wid * BATCH_PER_W

    pltpu.sync_copy(idxp_hbm.at[pl.ds(bbase, BATCH_PER_W)], idxp_v)

    def gather_desc(g, b):
        return pltpu.make_async_copy(
            tabpad_hbm.at[idxp_v.at[g]], rows_v.at[b], gsem)

    def scatter_desc(g, b):
        return pltpu.make_async_copy(rows_v.at[b], out_hbm.at[bbase + g], ssem)

    gather_desc(0, 0).start()

    def step(g, _):
        b = lax.rem(g, NBUF)
        gather_desc(g, b).wait()

        @pl.when(g >= 1)
        def _():
            scatter_desc(g - 1, 1 - b).wait()

        scatter_desc(g, b).start()

        @pl.when(g + 1 < BATCH_PER_W)
        def _():
            gather_desc(g + 1, 1 - b).start()

        return 0

    lax.fori_loop(0, BATCH_PER_W, step, 0)
    scatter_desc(BATCH_PER_W - 1, (BATCH_PER_W - 1) % NBUF).wait()


@functools.partial(
    pl.kernel,
    out_type=jax.ShapeDtypeStruct((B, TPAD, VOCAB), jnp.float32),
    mesh=plsc.VectorSubcoreMesh(core_axis_name="c", subcore_axis_name="s"),
    compiler_params=pltpu.CompilerParams(
        needs_layout_passes=False, use_tc_tiling_on_sc=False),
    scratch_types=[
        pltpu.VMEM((BATCH_PER_W, TPAD), jnp.int32),
        pltpu.VMEM((NBUF, TPAD, VOCAB), jnp.float32),
        pltpu.SemaphoreType.DMA,
        pltpu.SemaphoreType.DMA,
    ],
)
def _sc_gather(idxp_hbm, tabpad_hbm, out_hbm, idxp_v, rows_v, gsem, ssem):
    _gather_body(idxp_hbm, tabpad_hbm, out_hbm, idxp_v, rows_v, gsem, ssem)


def _loss_body(idx_hbm, tgt_hbm, lse_hbm, tabflat_hbm, part_hbm,
               idx_v, tgt_v, comb_v, picked_v, lse_v, acc_v, psem):
    cid = lax.axis_index("c")
    sid = lax.axis_index("s")
    wid = sid * NC + cid
    base = wid * TOK_PER_W

    pltpu.sync_copy(idx_hbm.at[pl.ds(base, TOK_PER_W)], idx_v)
    pltpu.sync_copy(tgt_hbm.at[pl.ds(base, TOK_PER_W)], tgt_v)
    pltpu.sync_copy(lse_hbm, lse_v)

    def comb_step(i, _):
        o = i * L
        comb_v[pl.ds(o, L)] = idx_v[pl.ds(o, L)] * VOCAB + tgt_v[pl.ds(o, L)]
        return 0
    lax.fori_loop(0, N_GROUPS, comb_step, 0)

    pltpu.async_copy(tabflat_hbm.at[comb_v], picked_v, psem).wait()

    def loss_step(i, acc):
        o = i * L
        idxv = idx_v[pl.ds(o, L)]
        lsev = plsc.load_gather(lse_v, [idxv])
        return acc + (lsev - picked_v[pl.ds(o, L)])

    acc = lax.fori_loop(0, N_GROUPS, loss_step, jnp.zeros((L,), jnp.float32))
    acc_v[...] = acc
    pltpu.sync_copy(acc_v, part_hbm.at[wid])


@functools.partial(
    pl.kernel,
    out_type=jax.ShapeDtypeStruct((NW, L), jnp.float32),
    mesh=plsc.VectorSubcoreMesh(core_axis_name="c", subcore_axis_name="s"),
    compiler_params=pltpu.CompilerParams(
        needs_layout_passes=False, use_tc_tiling_on_sc=False),
    scratch_types=[
        pltpu.VMEM((TOK_PER_W,), jnp.int32),
        pltpu.VMEM((TOK_PER_W,), jnp.int32),
        pltpu.VMEM((TOK_PER_W,), jnp.int32),
        pltpu.VMEM((TOK_PER_W,), jnp.float32),
        pltpu.VMEM((LSE_PAD,), jnp.float32),
        pltpu.VMEM((L,), jnp.float32),
        pltpu.SemaphoreType.DMA,
    ],
)
def _sc_loss(idx_hbm, tgt_hbm, lse_hbm, tabflat_hbm, part_hbm,
             idx_v, tgt_v, comb_v, picked_v, lse_v, acc_v, psem):
    _loss_body(idx_hbm, tgt_hbm, lse_hbm, tabflat_hbm, part_hbm,
               idx_v, tgt_v, comb_v, picked_v, lse_v, acc_v, psem)


def kernel(idx, targets, table):
    idx32 = idx.astype(jnp.int32)
    idx_flat = idx32.reshape(N_TOK)
    tgt_flat = targets.reshape(N_TOK).astype(jnp.int32)

    lse = _row_lse(table)
    tabflat = jnp.concatenate(
        [table.reshape(VOCAB * VOCAB), jnp.zeros((8,), jnp.float32)])
    parts = _sc_loss(idx_flat, tgt_flat, lse, tabflat)
    loss = jnp.sum(parts) / jnp.float32(N_TOK)

    junk = (jnp.arange(B)[:, None] * 7 + jnp.arange(TPAD - T)[None, :] * 131
            ) % VOCAB
    idxp = jnp.concatenate([idx32, junk.astype(jnp.int32)], axis=1)
    padded = _sc_gather(idxp, table)
    logits = padded[:, :T, :]
    return (logits, loss)
